# baseline (speedup 1.0000x reference)
import functools

import jax
import jax.numpy as jnp
from jax import lax
from jax.experimental import pallas as pl
from jax.experimental.pallas import tpu as pltpu
from jax.experimental.pallas import tpu_sc as plsc

NC, NS, L = 2, 16, 16
NW = NC * NS
B, SEQ, D = 4096, 50, 32
VOCAB = 1000000
QCH = 262144
TOK = 52
SPW = B // NW
GROUPS = SPW // L
CHUNK_S = 2
CHUNK_T = CHUNK_S * TOK
CHUNKS_PER_GROUP = L // CHUNK_S
GROUP_T = L * TOK

RL_C = 8192
RL_STEPS = QCH // RL_C


def _relayout_body(q0, q1, q2, q3, o_ref):
    o_ref[...] = jnp.concatenate(
        [q0[...].T, q1[...].T, q2[...].T, q3[...].T], axis=1
    )


def _q_spec(q):
    last = (VOCAB + RL_C - 1) // RL_C - 1

    def index_map(i):
        return (0, jnp.minimum(q * RL_STEPS + i, last))

    return pl.BlockSpec((D, RL_C), index_map)


def _pack_table(emb_t):
    return pl.pallas_call(
        _relayout_body,
        grid=(RL_STEPS,),
        in_specs=[_q_spec(0), _q_spec(1), _q_spec(2), _q_spec(3)],
        out_specs=pl.BlockSpec((RL_C, 4 * D), lambda i: (i, 0)),
        out_shape=jax.ShapeDtypeStruct((QCH, 4 * D), jnp.float32),
    )(emb_t, emb_t, emb_t, emb_t)


_MESH = plsc.VectorSubcoreMesh(
    core_axis_name="c", subcore_axis_name="s", num_cores=NC, num_subcores=NS
)


@functools.partial(
    pl.kernel,
    out_type=jax.ShapeDtypeStruct((B, D), jnp.float32),
    mesh=_MESH,
    compiler_params=pltpu.CompilerParams(use_tc_tiling_on_sc=False),
    scratch_types=[
        pltpu.VMEM((SPW * TOK,), jnp.int32),
        pltpu.VMEM((SPW,), jnp.int32),
        pltpu.VMEM((2, GROUP_T, D), jnp.float32),
        pltpu.VMEM((SPW, D), jnp.float32),
        pltpu.SemaphoreType.DMA,
        pltpu.SemaphoreType.DMA,
    ],
)
def _pooled_sc(x_hbm, len_hbm, tab_hbm, out_hbm, xw, lenw, rows, outb,
               sem0, sem1):
    wid = lax.axis_index("s") * NC + lax.axis_index("c")
    sbase = wid * SPW
    pltpu.sync_copy(x_hbm.at[pl.ds(wid * (SPW * TOK), SPW * TOK)], xw)
    pltpu.sync_copy(len_hbm.at[pl.ds(sbase, SPW)], lenw)

    def xform_body(i, carry):
        v = xw[pl.ds(i * L, L)]
        xw[pl.ds(i * L, L)] = ((v & (QCH - 1)) << 2) | ((v >> 18) & 3)
        return carry

    lax.fori_loop(0, SPW * TOK // L, xform_body, 0)

    sems = (sem0, sem1)

    def issue(g, buf):
        descs = []
        for k in range(CHUNKS_PER_GROUP):
            off = (g * L + k * CHUNK_S) * TOK
            descs.append(
                pltpu.async_copy(
                    tab_hbm.at[xw.at[pl.ds(off, CHUNK_T)]],
                    rows.at[buf, pl.ds(k * CHUNK_T, CHUNK_T)],
                    sems[buf],
                )
            )
        return descs

    pending = {0: issue(0, 0)}
    for g in range(GROUPS):
        buf = g % 2
        if g + 1 < GROUPS:
            pending[1 - buf] = issue(g + 1, 1 - buf)
        for d_ in pending[buf]:
            d_.wait()
        len16f = lenw[pl.ds(g * L, L)].astype(jnp.float32)
        zf = jnp.zeros((L,), jnp.float32)

        def sample_body(i, carry, buf=buf, len16f=len16f, zf=zf):
            iv = jnp.broadcast_to(i, (L,)).astype(jnp.int32)
            lfv = len16f.at[iv].get(mode="promise_in_bounds")
            base = i * TOK
            lo = zf
            hi = zf
            for j in range(SEQ):
                mf = jnp.clip(lfv - jnp.float32(j), 0.0, 1.0)
                lo = lo + rows[buf, base + j, pl.ds(0, L)] * mf
                hi = hi + rows[buf, base + j, pl.ds(L, L)] * mf
            s = g * L + i
            outb[s, pl.ds(0, L)] = jnp.maximum(lo / lfv, 0.0)
            outb[s, pl.ds(L, L)] = jnp.maximum(hi / lfv, 0.0)
            return carry

        lax.fori_loop(0, L, sample_body, 0)
    pltpu.sync_copy(outb, out_hbm.at[pl.ds(sbase, SPW)])


def _head_body(p_ref, w_ref, b_ref, o_ref):
    o_ref[...] = (
        jnp.dot(p_ref[...], w_ref[...], preferred_element_type=jnp.float32)
        + b_ref[...]
    )


def _tc_head(pooled, W, b2):
    return pl.pallas_call(
        _head_body,
        out_shape=jax.ShapeDtypeStruct((B, W.shape[1]), jnp.float32),
    )(pooled, W, b2)


def kernel(x, lengths, emb_table, W, b):
    tabp = (
        jnp.pad(emb_table.T, ((0, 0), (0, 4 * QCH - VOCAB)))
        .reshape(D, 4, QCH).transpose(2, 1, 0).reshape(QCH, 4 * D)
    )
    tab2 = tabp.reshape(4 * QCH, D)
    x_flat = jnp.pad(x, ((0, 0), (0, TOK - SEQ))).reshape(-1)
    pooled = _pooled_sc(x_flat, lengths, tab2)
    return _tc_head(pooled, W, b.reshape(1, -1))


# speedup vs baseline: 2.7420x; 2.7420x over previous
import functools

import jax
import jax.numpy as jnp
from jax import lax
from jax.experimental import pallas as pl
from jax.experimental.pallas import tpu as pltpu
from jax.experimental.pallas import tpu_sc as plsc

NC, NS, L = 2, 16, 16
NW = NC * NS
B, SEQ, D = 4096, 50, 32
VOCAB = 1000000
QCH = 262144
TOK = 52
SPW = B // NW
GROUPS = SPW // L
CHUNK_S = 2
CHUNK_T = CHUNK_S * TOK
CHUNKS_PER_GROUP = L // CHUNK_S
GROUP_T = L * TOK

RL_C = 8192
RL_STEPS = QCH // RL_C


def _relayout_body(q0, q1, q2, q3, o_ref):
    for q, ref in enumerate((q0, q1, q2, q3)):
        o_ref[:, q * D:(q + 1) * D] = ref[...].T


def _q_spec(q):
    last = (VOCAB + RL_C - 1) // RL_C - 1

    def index_map(i):
        return (0, jnp.minimum(q * RL_STEPS + i, last))

    return pl.BlockSpec((D, RL_C), index_map)


def _pack_table(emb_t):
    return pl.pallas_call(
        _relayout_body,
        grid=(RL_STEPS,),
        in_specs=[_q_spec(0), _q_spec(1), _q_spec(2), _q_spec(3)],
        out_specs=pl.BlockSpec((RL_C, 4 * D), lambda i: (i, 0)),
        out_shape=jax.ShapeDtypeStruct((QCH, 4 * D), jnp.float32),
    )(emb_t, emb_t, emb_t, emb_t)


_MESH = plsc.VectorSubcoreMesh(
    core_axis_name="c", subcore_axis_name="s", num_cores=NC, num_subcores=NS
)


@functools.partial(
    pl.kernel,
    out_type=jax.ShapeDtypeStruct((B, D), jnp.float32),
    mesh=_MESH,
    compiler_params=pltpu.CompilerParams(use_tc_tiling_on_sc=False),
    scratch_types=[
        pltpu.VMEM((SPW * TOK,), jnp.int32),
        pltpu.VMEM((SPW,), jnp.int32),
        pltpu.VMEM((2, GROUP_T, D), jnp.float32),
        pltpu.VMEM((SPW, D), jnp.float32),
        pltpu.SemaphoreType.DMA,
        pltpu.SemaphoreType.DMA,
    ],
)
def _pooled_sc(x_hbm, len_hbm, tab_hbm, out_hbm, xw, lenw, rows, outb,
               sem0, sem1):
    wid = lax.axis_index("s") * NC + lax.axis_index("c")
    sbase = wid * SPW
    pltpu.sync_copy(x_hbm.at[pl.ds(wid * (SPW * TOK), SPW * TOK)], xw)
    pltpu.sync_copy(len_hbm.at[pl.ds(sbase, SPW)], lenw)

    def xform_body(i, carry):
        v = xw[pl.ds(i * L, L)]
        xw[pl.ds(i * L, L)] = ((v & (QCH - 1)) << 2) | ((v >> 18) & 3)
        return carry

    lax.fori_loop(0, SPW * TOK // L, xform_body, 0)

    sems = (sem0, sem1)

    def issue(g, buf):
        descs = []
        for k in range(CHUNKS_PER_GROUP):
            off = (g * L + k * CHUNK_S) * TOK
            descs.append(
                pltpu.async_copy(
                    tab_hbm.at[xw.at[pl.ds(off, CHUNK_T)]],
                    rows.at[buf, pl.ds(k * CHUNK_T, CHUNK_T)],
                    sems[buf],
                )
            )
        return descs

    pending = {0: issue(0, 0)}
    for g in range(GROUPS):
        buf = g % 2
        if g + 1 < GROUPS:
            pending[1 - buf] = issue(g + 1, 1 - buf)
        for d_ in pending[buf]:
            d_.wait()
        len16f = lenw[pl.ds(g * L, L)].astype(jnp.float32)
        zf = jnp.zeros((L,), jnp.float32)

        def sample_body(i, carry, buf=buf, len16f=len16f, zf=zf):
            iv = jnp.broadcast_to(i, (L,)).astype(jnp.int32)
            lfv = len16f.at[iv].get(mode="promise_in_bounds")
            base = i * TOK
            lo = [zf, zf, zf, zf]
            hi = [zf, zf, zf, zf]
            for j in range(SEQ):
                mf = jnp.clip(lfv - jnp.float32(j), 0.0, 1.0)
                a = j % 4
                lo[a] = lo[a] + rows[buf, base + j, pl.ds(0, L)] * mf
                hi[a] = hi[a] + rows[buf, base + j, pl.ds(L, L)] * mf
            los = (lo[0] + lo[1]) + (lo[2] + lo[3])
            his = (hi[0] + hi[1]) + (hi[2] + hi[3])
            s = g * L + i
            outb[s, pl.ds(0, L)] = jnp.maximum(los / lfv, 0.0)
            outb[s, pl.ds(L, L)] = jnp.maximum(his / lfv, 0.0)
            return carry

        lax.fori_loop(0, L, sample_body, 0)
    pltpu.sync_copy(outb, out_hbm.at[pl.ds(sbase, SPW)])


def _head_body(p_ref, w_ref, b_ref, o_ref):
    o_ref[...] = (
        jnp.dot(p_ref[...], w_ref[...], preferred_element_type=jnp.float32)
        + b_ref[...]
    )


def _tc_head(pooled, W, b2):
    return pl.pallas_call(
        _head_body,
        out_shape=jax.ShapeDtypeStruct((B, W.shape[1]), jnp.float32),
    )(pooled, W, b2)


def kernel(x, lengths, emb_table, W, b):
    tabp = _pack_table(emb_table.T)
    tab2 = tabp.reshape(4 * QCH, D)
    x_flat = jnp.pad(x, ((0, 0), (0, TOK - SEQ))).reshape(-1)
    pooled = _pooled_sc(x_flat, lengths, tab2)
    return _tc_head(pooled, W, b.reshape(1, -1))


# speedup vs baseline: 2.7555x; 1.0049x over previous
import functools

import jax
import jax.numpy as jnp
from jax import lax
from jax.experimental import pallas as pl
from jax.experimental.pallas import tpu as pltpu
from jax.experimental.pallas import tpu_sc as plsc

NC, NS, L = 2, 16, 16
NW = NC * NS
B, SEQ, D = 4096, 50, 32
VOCAB = 1000000
QCH = 262144
TOK = 52
SPW = B // NW
GROUPS = SPW // L
CHUNK_S = 2
CHUNK_T = CHUNK_S * TOK
CHUNKS_PER_GROUP = L // CHUNK_S
GROUP_T = L * TOK

RL_C = 8192
RL_STEPS = QCH // RL_C


def _relayout_body(q0, q1, q2, q3, o_ref):
    for q, ref in enumerate((q0, q1, q2, q3)):
        o_ref[:, q * D:(q + 1) * D] = ref[...].T


def _q_spec(q):
    last = (VOCAB + RL_C - 1) // RL_C - 1

    def index_map(i):
        return (0, jnp.minimum(q * RL_STEPS + i, last))

    return pl.BlockSpec((D, RL_C), index_map)


def _pack_table(emb_t):
    return pl.pallas_call(
        _relayout_body,
        grid=(RL_STEPS,),
        in_specs=[_q_spec(0), _q_spec(1), _q_spec(2), _q_spec(3)],
        out_specs=pl.BlockSpec((RL_C, 4 * D), lambda i: (i, 0)),
        out_shape=jax.ShapeDtypeStruct((QCH, 4 * D), jnp.float32),
    )(emb_t, emb_t, emb_t, emb_t)


_MESH = plsc.VectorSubcoreMesh(
    core_axis_name="c", subcore_axis_name="s", num_cores=NC, num_subcores=NS
)


@functools.partial(
    pl.kernel,
    out_type=jax.ShapeDtypeStruct((B, D), jnp.float32),
    mesh=_MESH,
    compiler_params=pltpu.CompilerParams(use_tc_tiling_on_sc=False),
    scratch_types=[
        pltpu.VMEM((SPW * TOK,), jnp.int32),
        pltpu.VMEM((SPW,), jnp.int32),
        pltpu.VMEM((3, GROUP_T, D), jnp.float32),
        pltpu.VMEM((SPW, D), jnp.float32),
        pltpu.SemaphoreType.DMA,
        pltpu.SemaphoreType.DMA,
        pltpu.SemaphoreType.DMA,
    ],
)
def _pooled_sc(x_hbm, len_hbm, tab_hbm, out_hbm, xw, lenw, rows, outb,
               sem0, sem1, sem2):
    wid = lax.axis_index("s") * NC + lax.axis_index("c")
    sbase = wid * SPW
    pltpu.sync_copy(x_hbm.at[pl.ds(wid * (SPW * TOK), SPW * TOK)], xw)
    pltpu.sync_copy(len_hbm.at[pl.ds(sbase, SPW)], lenw)

    def xform_body(i, carry):
        v = xw[pl.ds(i * L, L)]
        xw[pl.ds(i * L, L)] = ((v & (QCH - 1)) << 2) | ((v >> 18) & 3)
        return carry

    lax.fori_loop(0, SPW * TOK // L, xform_body, 0)

    sems = (sem0, sem1, sem2)

    def issue(g, buf):
        descs = []
        for k in range(CHUNKS_PER_GROUP):
            off = (g * L + k * CHUNK_S) * TOK
            descs.append(
                pltpu.async_copy(
                    tab_hbm.at[xw.at[pl.ds(off, CHUNK_T)]],
                    rows.at[buf, pl.ds(k * CHUNK_T, CHUNK_T)],
                    sems[buf],
                )
            )
        return descs

    pending = {0: issue(0, 0), 1: issue(1, 1)}
    for g in range(GROUPS):
        buf = g % 3
        if g + 2 < GROUPS:
            pending[(g + 2) % 3] = issue(g + 2, (g + 2) % 3)
        for d_ in pending[buf]:
            d_.wait()
        len16f = lenw[pl.ds(g * L, L)].astype(jnp.float32)
        zf = jnp.zeros((L,), jnp.float32)

        def sample_body(i, carry, buf=buf, len16f=len16f, zf=zf):
            iv = jnp.broadcast_to(i, (L,)).astype(jnp.int32)
            lfv = len16f.at[iv].get(mode="promise_in_bounds")
            base = i * TOK
            lo = [zf, zf, zf, zf]
            hi = [zf, zf, zf, zf]
            for j in range(SEQ):
                mf = jnp.clip(lfv - jnp.float32(j), 0.0, 1.0)
                a = j % 4
                lo[a] = lo[a] + rows[buf, base + j, pl.ds(0, L)] * mf
                hi[a] = hi[a] + rows[buf, base + j, pl.ds(L, L)] * mf
            los = (lo[0] + lo[1]) + (lo[2] + lo[3])
            his = (hi[0] + hi[1]) + (hi[2] + hi[3])
            s = g * L + i
            outb[s, pl.ds(0, L)] = jnp.maximum(los / lfv, 0.0)
            outb[s, pl.ds(L, L)] = jnp.maximum(his / lfv, 0.0)
            return carry

        lax.fori_loop(0, L, sample_body, 0)
    pltpu.sync_copy(outb, out_hbm.at[pl.ds(sbase, SPW)])


def _head_body(p_ref, w_ref, b_ref, o_ref):
    o_ref[...] = (
        jnp.dot(p_ref[...], w_ref[...], preferred_element_type=jnp.float32)
        + b_ref[...]
    )


def _tc_head(pooled, W, b2):
    return pl.pallas_call(
        _head_body,
        out_shape=jax.ShapeDtypeStruct((B, W.shape[1]), jnp.float32),
    )(pooled, W, b2)


def kernel(x, lengths, emb_table, W, b):
    tabp = _pack_table(emb_table.T)
    tab2 = tabp.reshape(4 * QCH, D)
    x_flat = jnp.pad(x, ((0, 0), (0, TOK - SEQ))).reshape(-1)
    pooled = _pooled_sc(x_flat, lengths, tab2)
    return _tc_head(pooled, W, b.reshape(1, -1))
